# Initial kernel scaffold; baseline (speedup 1.0000x reference)
import jax
import jax.numpy as jnp
from jax.experimental import pallas as pl


def kernel(x, pos, Wd, g1, b1, W1, g2, b2, W2, g3, b3):
    raise NotImplementedError("write your pallas kernel here")


# speedup vs baseline: 1.0250x; 1.0250x over previous
import functools

import jax
import jax.numpy as jnp
from jax.experimental import pallas as pl
from jax.experimental.pallas import tpu as pltpu

_BN_EPS = 1e-5
_NEG = 0.2


def _lrelu(v):
    return jnp.where(v >= 0, v, _NEG * v)


def _mm_kernel(x_ref, w_ref, o_ref):
    o_ref[...] = jnp.dot(x_ref[...], w_ref[...],
                         preferred_element_type=jnp.float32)


def _matmul(x, w):
    m, k = x.shape
    k2, n = w.shape
    return pl.pallas_call(
        _mm_kernel,
        out_shape=jax.ShapeDtypeStruct((m, n), jnp.float32),
    )(x, w)


def _edge1_kernel(g_ref, q_ref, ymax_ref, stat_ref, *, kk, cc):
    r = pl.program_id(0)
    q = q_ref[...]
    acc_s = jnp.zeros((1, cc), jnp.float32)
    acc_q = jnp.zeros((1, cc), jnp.float32)
    m = jnp.full(q.shape, -jnp.inf, jnp.float32)
    for k in range(kk):
        y = g_ref[:, k * cc:(k + 1) * cc] + q
        acc_s = acc_s + jnp.sum(y, axis=0, keepdims=True)
        acc_q = acc_q + jnp.sum(y * y, axis=0, keepdims=True)
        m = jnp.maximum(m, y)
    ymax_ref[...] = m

    @pl.when(r == 0)
    def _init():
        stat_ref[...] = jnp.zeros_like(stat_ref)

    stat_ref[0:1, :] = stat_ref[0:1, :] + acc_s
    stat_ref[1:2, :] = stat_ref[1:2, :] + acc_q


def _edge1(g, q, kk, cc, rows_tile):
    rows = g.shape[0]
    grid = rows // rows_tile
    return pl.pallas_call(
        functools.partial(_edge1_kernel, kk=kk, cc=cc),
        grid=(grid,),
        in_specs=[
            pl.BlockSpec((rows_tile, kk * cc), lambda i: (i, 0)),
            pl.BlockSpec((rows_tile, cc), lambda i: (i, 0)),
        ],
        out_specs=[
            pl.BlockSpec((rows_tile, cc), lambda i: (i, 0)),
            pl.BlockSpec((8, cc), lambda i: (0, 0)),
        ],
        out_shape=[
            jax.ShapeDtypeStruct((rows, cc), jnp.float32),
            jax.ShapeDtypeStruct((8, cc), jnp.float32),
        ],
    )(g, q)


def _fin1_mm_kernel(ymax_ref, stat_ref, gb_ref, w_ref, o_ref, *, cnt):
    mean = stat_ref[0:1, :] / cnt
    var = stat_ref[1:2, :] / cnt - mean * mean
    scale = gb_ref[0:1, :] * jax.lax.rsqrt(var + _BN_EPS)
    shift = gb_ref[1:2, :] - mean * scale
    o4 = _lrelu(ymax_ref[...] * scale + shift)
    o_ref[...] = jnp.dot(o4, w_ref[...], preferred_element_type=jnp.float32)


def _fin1_mm(ymax, stat, gb, w, cnt):
    rows, cc = ymax.shape
    n = w.shape[1]
    return pl.pallas_call(
        functools.partial(_fin1_mm_kernel, cnt=cnt),
        out_shape=jax.ShapeDtypeStruct((rows, n), jnp.float32),
    )(ymax, stat, gb, w)


def _s2a_kernel(gp_ref, q_ref, c1_ref, stat_ref, *, kk, cc):
    r = pl.program_id(0)
    q = q_ref[...]
    acc_s = jnp.zeros((1, cc), jnp.float32)
    acc_q = jnp.zeros((1, cc), jnp.float32)
    for k in range(kk):
        c = gp_ref[:, k * cc:(k + 1) * cc] + q
        c1_ref[:, k * cc:(k + 1) * cc] = c
        acc_s = acc_s + jnp.sum(c, axis=0, keepdims=True)
        acc_q = acc_q + jnp.sum(c * c, axis=0, keepdims=True)

    @pl.when(r == 0)
    def _init():
        stat_ref[...] = jnp.zeros_like(stat_ref)

    stat_ref[0:1, :] = stat_ref[0:1, :] + acc_s
    stat_ref[1:2, :] = stat_ref[1:2, :] + acc_q


def _s2a(gp, q, kk, cc, rows_tile):
    rows = gp.shape[0]
    grid = rows // rows_tile
    return pl.pallas_call(
        functools.partial(_s2a_kernel, kk=kk, cc=cc),
        grid=(grid,),
        in_specs=[
            pl.BlockSpec((rows_tile, kk * cc), lambda i: (i, 0)),
            pl.BlockSpec((rows_tile, cc), lambda i: (i, 0)),
        ],
        out_specs=[
            pl.BlockSpec((rows_tile, kk * cc), lambda i: (i, 0)),
            pl.BlockSpec((8, cc), lambda i: (0, 0)),
        ],
        out_shape=[
            jax.ShapeDtypeStruct((rows, kk * cc), jnp.float32),
            jax.ShapeDtypeStruct((8, cc), jnp.float32),
        ],
    )(gp, q)


def _s2b_kernel(c1_ref, stat1_ref, gb2_ref, w2_ref, r_ref, stat2_ref,
                *, kk, cc, cnt1):
    r = pl.program_id(0)
    mean = stat1_ref[0:1, :] / cnt1
    var = stat1_ref[1:2, :] / cnt1 - mean * mean
    scale = gb2_ref[0:1, :] * jax.lax.rsqrt(var + _BN_EPS)
    shift = gb2_ref[1:2, :] - mean * scale
    w2 = w2_ref[...]
    acc_s = jnp.zeros((1, cc), jnp.float32)
    acc_q = jnp.zeros((1, cc), jnp.float32)
    m = jnp.full((c1_ref.shape[0], cc), -jnp.inf, jnp.float32)
    for k in range(kk):
        h = _lrelu(c1_ref[:, k * cc:(k + 1) * cc] * scale + shift)
        c2 = jnp.dot(h, w2, preferred_element_type=jnp.float32)
        acc_s = acc_s + jnp.sum(c2, axis=0, keepdims=True)
        acc_q = acc_q + jnp.sum(c2 * c2, axis=0, keepdims=True)
        m = jnp.maximum(m, c2)
    r_ref[...] = m

    @pl.when(r == 0)
    def _init():
        stat2_ref[...] = jnp.zeros_like(stat2_ref)

    stat2_ref[0:1, :] = stat2_ref[0:1, :] + acc_s
    stat2_ref[1:2, :] = stat2_ref[1:2, :] + acc_q


def _s2b(c1, stat1, gb2, w2t, kk, cc, cnt1, rows_tile):
    rows = c1.shape[0]
    grid = rows // rows_tile
    return pl.pallas_call(
        functools.partial(_s2b_kernel, kk=kk, cc=cc, cnt1=cnt1),
        grid=(grid,),
        in_specs=[
            pl.BlockSpec((rows_tile, kk * cc), lambda i: (i, 0)),
            pl.BlockSpec((8, cc), lambda i: (0, 0)),
            pl.BlockSpec((2, cc), lambda i: (0, 0)),
            pl.BlockSpec((cc, cc), lambda i: (0, 0)),
        ],
        out_specs=[
            pl.BlockSpec((rows_tile, cc), lambda i: (i, 0)),
            pl.BlockSpec((8, cc), lambda i: (0, 0)),
        ],
        out_shape=[
            jax.ShapeDtypeStruct((rows, cc), jnp.float32),
            jax.ShapeDtypeStruct((8, cc), jnp.float32),
        ],
    )(c1, stat1, gb2, w2t)


def _fin2_kernel(r_ref, stat_ref, gb_ref, o_ref, *, cnt):
    mean = stat_ref[0:1, :] / cnt
    var = stat_ref[1:2, :] / cnt - mean * mean
    scale = gb_ref[0:1, :] * jax.lax.rsqrt(var + _BN_EPS)
    shift = gb_ref[1:2, :] - mean * scale
    o_ref[...] = _lrelu(r_ref[...] * scale + shift)


def _fin2(rr, stat, gb, cnt):
    return pl.pallas_call(
        functools.partial(_fin2_kernel, cnt=cnt),
        out_shape=jax.ShapeDtypeStruct(rr.shape, jnp.float32),
    )(rr, stat, gb)


def kernel(x, pos, Wd, g1, b1, W1, g2, b2, W2, g3, b3):
    B, N, C = x.shape
    ALLK, KD, KDP, KFPS = 128, 8, 6, 32

    sqp = jnp.sum(pos * pos, axis=-1)
    dp2 = (sqp[:, :, None] + sqp[:, None, :]
           - 2.0 * jnp.einsum('bnd,bmd->bnm', pos, pos))
    dp = jnp.sqrt(jnp.maximum(dp2, 1e-12))
    idx_l = jax.lax.top_k(-dp, ALLK)[1]
    idx_pos = idx_l[:, :, 1:KDP + 1]

    sqx = jnp.sum(x * x, axis=-1)
    dx2 = (sqx[:, :, None] + sqx[:, None, :]
           - 2.0 * jnp.einsum('bnd,bmd->bnm', x, x))
    dx = jnp.sqrt(jnp.maximum(dx2, 1e-12))
    idx_feat = jax.lax.top_k(-dx, KD + 1)[1][:, :, 1:]
    idx_out = jnp.concatenate([idx_feat, idx_pos], axis=2)

    R = B * N
    idx_l_flat = idx_l.reshape(R, ALLK)
    pos_flat = pos.reshape(R, 3)
    neigh = pos_flat[idx_l_flat]
    distance = jnp.full((R, ALLK), 1e10, jnp.float32)
    far = jnp.zeros((R,), jnp.int32)
    ridx = jnp.arange(R)
    cents = []
    for _ in range(KFPS):
        cents.append(far)
        centroid = neigh[ridx, far][:, None, :]
        dist = jnp.sum((neigh - centroid) ** 2, axis=-1)
        distance = jnp.minimum(distance, dist)
        far = jnp.argmax(distance, axis=-1).astype(jnp.int32)
    cents = jnp.stack(cents, axis=1)
    idx_fps = jnp.take_along_axis(idx_l_flat, cents, axis=1)

    C2 = 2 * C
    x2d = x.reshape(R, C)
    A = Wd[:, :C]
    Bm = Wd[:, C:]
    wcat1 = jnp.concatenate([A.T, (Bm - A).T], axis=1)
    pq = _matmul(x2d, wcat1)
    px, qx = pq[:, :C2], pq[:, C2:]

    flat14 = (idx_out + jnp.arange(B)[:, None, None] * N).reshape(-1)
    g14 = px[flat14].reshape(R, 14 * C2)
    ymax, stat1 = _edge1(g14, qx, 14, C2, 512)

    gb1 = jnp.stack([g1, b1], axis=0)
    A1 = W1[:, :C2]
    B1 = W1[:, C2:]
    wcat2 = jnp.concatenate([A1.T, (B1 - A1).T], axis=1)
    pq2 = _fin1_mm(ymax, stat1, gb1, wcat2, float(R * 14))
    p2, q2 = pq2[:, :128], pq2[:, 128:]

    flat32 = (idx_fps.reshape(B, N, KFPS)
              + jnp.arange(B)[:, None, None] * N).reshape(-1)
    gp = p2[flat32].reshape(R, KFPS * 128)
    c1, s2stat1 = _s2a(gp, q2, KFPS, 128, 256)

    gb2 = jnp.stack([g2, b2], axis=0)
    rr, s2stat2 = _s2b(c1, s2stat1, gb2, W2.T, KFPS, 128,
                       float(R * KFPS), 256)

    gb3 = jnp.stack([g3, b3], axis=0)
    out = _fin2(rr, s2stat2, gb3, float(R * KFPS)).reshape(B, N, 128)
    return out, idx_out


# speedup vs baseline: 1.2866x; 1.2553x over previous
import functools

import jax
import jax.numpy as jnp
from jax.experimental import pallas as pl
from jax.experimental.pallas import tpu as pltpu

_BN_EPS = 1e-5
_NEG = 0.2


def _lrelu(v):
    return jnp.where(v >= 0, v, _NEG * v)


def _mm_kernel(x_ref, w_ref, o_ref):
    o_ref[...] = jnp.dot(x_ref[...], w_ref[...],
                         preferred_element_type=jnp.float32)


def _matmul(x, w):
    m, k = x.shape
    k2, n = w.shape
    return pl.pallas_call(
        _mm_kernel,
        out_shape=jax.ShapeDtypeStruct((m, n), jnp.float32),
    )(x, w)


def _edge1_kernel(g_ref, q_ref, ymax_ref, stat_ref, *, kk, cc):
    r = pl.program_id(0)
    q = q_ref[...]
    acc_s = jnp.zeros((1, cc), jnp.float32)
    acc_q = jnp.zeros((1, cc), jnp.float32)
    m = jnp.full(q.shape, -jnp.inf, jnp.float32)
    for k in range(kk):
        y = g_ref[:, k * cc:(k + 1) * cc] + q
        acc_s = acc_s + jnp.sum(y, axis=0, keepdims=True)
        acc_q = acc_q + jnp.sum(y * y, axis=0, keepdims=True)
        m = jnp.maximum(m, y)
    ymax_ref[...] = m

    @pl.when(r == 0)
    def _init():
        stat_ref[...] = jnp.zeros_like(stat_ref)

    stat_ref[0:1, :] = stat_ref[0:1, :] + acc_s
    stat_ref[1:2, :] = stat_ref[1:2, :] + acc_q


def _edge1(g, q, kk, cc, rows_tile):
    rows = g.shape[0]
    grid = rows // rows_tile
    return pl.pallas_call(
        functools.partial(_edge1_kernel, kk=kk, cc=cc),
        grid=(grid,),
        in_specs=[
            pl.BlockSpec((rows_tile, kk * cc), lambda i: (i, 0)),
            pl.BlockSpec((rows_tile, cc), lambda i: (i, 0)),
        ],
        out_specs=[
            pl.BlockSpec((rows_tile, cc), lambda i: (i, 0)),
            pl.BlockSpec((8, cc), lambda i: (0, 0)),
        ],
        out_shape=[
            jax.ShapeDtypeStruct((rows, cc), jnp.float32),
            jax.ShapeDtypeStruct((8, cc), jnp.float32),
        ],
    )(g, q)


def _fin1_mm_kernel(ymax_ref, stat_ref, gb_ref, w_ref, o_ref, *, cnt):
    mean = stat_ref[0:1, :] / cnt
    var = stat_ref[1:2, :] / cnt - mean * mean
    scale = gb_ref[0:1, :] * jax.lax.rsqrt(var + _BN_EPS)
    shift = gb_ref[1:2, :] - mean * scale
    o4 = _lrelu(ymax_ref[...] * scale + shift)
    o_ref[...] = jnp.dot(o4, w_ref[...], preferred_element_type=jnp.float32)


def _fin1_mm(ymax, stat, gb, w, cnt):
    rows, cc = ymax.shape
    n = w.shape[1]
    return pl.pallas_call(
        functools.partial(_fin1_mm_kernel, cnt=cnt),
        out_shape=jax.ShapeDtypeStruct((rows, n), jnp.float32),
    )(ymax, stat, gb, w)


def _s2a_kernel(gp_ref, q_ref, c1_ref, stat_ref, *, kk, cc):
    r = pl.program_id(0)
    q = q_ref[...]
    acc_s = jnp.zeros((1, cc), jnp.float32)
    acc_q = jnp.zeros((1, cc), jnp.float32)
    for k in range(kk):
        c = gp_ref[:, k * cc:(k + 1) * cc] + q
        c1_ref[:, k * cc:(k + 1) * cc] = c
        acc_s = acc_s + jnp.sum(c, axis=0, keepdims=True)
        acc_q = acc_q + jnp.sum(c * c, axis=0, keepdims=True)

    @pl.when(r == 0)
    def _init():
        stat_ref[...] = jnp.zeros_like(stat_ref)

    stat_ref[0:1, :] = stat_ref[0:1, :] + acc_s
    stat_ref[1:2, :] = stat_ref[1:2, :] + acc_q


def _s2a(gp, q, kk, cc, rows_tile):
    rows = gp.shape[0]
    grid = rows // rows_tile
    return pl.pallas_call(
        functools.partial(_s2a_kernel, kk=kk, cc=cc),
        grid=(grid,),
        in_specs=[
            pl.BlockSpec((rows_tile, kk * cc), lambda i: (i, 0)),
            pl.BlockSpec((rows_tile, cc), lambda i: (i, 0)),
        ],
        out_specs=[
            pl.BlockSpec((rows_tile, kk * cc), lambda i: (i, 0)),
            pl.BlockSpec((8, cc), lambda i: (0, 0)),
        ],
        out_shape=[
            jax.ShapeDtypeStruct((rows, kk * cc), jnp.float32),
            jax.ShapeDtypeStruct((8, cc), jnp.float32),
        ],
    )(gp, q)


def _s2b_kernel(c1_ref, stat1_ref, gb2_ref, w2_ref, r_ref, stat2_ref,
                *, kk, cc, cnt1):
    r = pl.program_id(0)
    mean = stat1_ref[0:1, :] / cnt1
    var = stat1_ref[1:2, :] / cnt1 - mean * mean
    scale = gb2_ref[0:1, :] * jax.lax.rsqrt(var + _BN_EPS)
    shift = gb2_ref[1:2, :] - mean * scale
    w2 = w2_ref[...]
    acc_s = jnp.zeros((1, cc), jnp.float32)
    acc_q = jnp.zeros((1, cc), jnp.float32)
    m = jnp.full((c1_ref.shape[0], cc), -jnp.inf, jnp.float32)
    for k in range(kk):
        h = _lrelu(c1_ref[:, k * cc:(k + 1) * cc] * scale + shift)
        c2 = jnp.dot(h, w2, preferred_element_type=jnp.float32)
        acc_s = acc_s + jnp.sum(c2, axis=0, keepdims=True)
        acc_q = acc_q + jnp.sum(c2 * c2, axis=0, keepdims=True)
        m = jnp.maximum(m, c2)
    r_ref[...] = m

    @pl.when(r == 0)
    def _init():
        stat2_ref[...] = jnp.zeros_like(stat2_ref)

    stat2_ref[0:1, :] = stat2_ref[0:1, :] + acc_s
    stat2_ref[1:2, :] = stat2_ref[1:2, :] + acc_q


def _s2b(c1, stat1, gb2, w2t, kk, cc, cnt1, rows_tile):
    rows = c1.shape[0]
    grid = rows // rows_tile
    return pl.pallas_call(
        functools.partial(_s2b_kernel, kk=kk, cc=cc, cnt1=cnt1),
        grid=(grid,),
        in_specs=[
            pl.BlockSpec((rows_tile, kk * cc), lambda i: (i, 0)),
            pl.BlockSpec((8, cc), lambda i: (0, 0)),
            pl.BlockSpec((2, cc), lambda i: (0, 0)),
            pl.BlockSpec((cc, cc), lambda i: (0, 0)),
        ],
        out_specs=[
            pl.BlockSpec((rows_tile, cc), lambda i: (i, 0)),
            pl.BlockSpec((8, cc), lambda i: (0, 0)),
        ],
        out_shape=[
            jax.ShapeDtypeStruct((rows, cc), jnp.float32),
            jax.ShapeDtypeStruct((8, cc), jnp.float32),
        ],
    )(c1, stat1, gb2, w2t)


def _fin2_kernel(r_ref, stat_ref, gb_ref, o_ref, *, cnt):
    mean = stat_ref[0:1, :] / cnt
    var = stat_ref[1:2, :] / cnt - mean * mean
    scale = gb_ref[0:1, :] * jax.lax.rsqrt(var + _BN_EPS)
    shift = gb_ref[1:2, :] - mean * scale
    o_ref[...] = _lrelu(r_ref[...] * scale + shift)


def _fin2(rr, stat, gb, cnt):
    return pl.pallas_call(
        functools.partial(_fin2_kernel, cnt=cnt),
        out_shape=jax.ShapeDtypeStruct(rr.shape, jnp.float32),
    )(rr, stat, gb)


def _fps_kernel(nx_ref, ny_ref, nz_ref, cents_ref, *, npoint, allk):
    nx = nx_ref[...]
    ny = ny_ref[...]
    nz = nz_ref[...]
    rows = nx.shape[0]
    lane = jax.lax.broadcasted_iota(jnp.int32, (rows, allk), 1).astype(jnp.float32)
    dist = jnp.full((rows, allk), 1e10, jnp.float32)
    sel = jnp.zeros((rows, 1), jnp.float32)
    for t in range(npoint):
        cents_ref[:, t:t + 1] = sel.astype(jnp.int32)
        oh = (lane == sel).astype(jnp.float32)
        cx = jnp.sum(nx * oh, axis=1, keepdims=True)
        cy = jnp.sum(ny * oh, axis=1, keepdims=True)
        cz = jnp.sum(nz * oh, axis=1, keepdims=True)
        dx = nx - cx
        dy = ny - cy
        dz = nz - cz
        d = dx * dx + dy * dy + dz * dz
        dist = jnp.minimum(dist, d)
        mx = jnp.max(dist, axis=1, keepdims=True)
        sel = jnp.min(jnp.where(dist == mx, lane, float(allk)),
                      axis=1, keepdims=True)


def _fps(nx, ny, nz, npoint, rows_tile):
    rows, allk = nx.shape
    grid = rows // rows_tile
    spec = pl.BlockSpec((rows_tile, allk), lambda i: (i, 0))
    return pl.pallas_call(
        functools.partial(_fps_kernel, npoint=npoint, allk=allk),
        grid=(grid,),
        in_specs=[spec, spec, spec],
        out_specs=pl.BlockSpec((rows_tile, npoint), lambda i: (i, 0)),
        out_shape=jax.ShapeDtypeStruct((rows, npoint), jnp.int32),
    )(nx, ny, nz)


def kernel(x, pos, Wd, g1, b1, W1, g2, b2, W2, g3, b3):
    B, N, C = x.shape
    ALLK, KD, KDP, KFPS = 128, 8, 6, 32

    sqp = jnp.sum(pos * pos, axis=-1)
    dp2 = (sqp[:, :, None] + sqp[:, None, :]
           - 2.0 * jnp.einsum('bnd,bmd->bnm', pos, pos))
    dp = jnp.sqrt(jnp.maximum(dp2, 1e-12))
    idx_l = jax.lax.top_k(-dp, ALLK)[1]
    idx_pos = idx_l[:, :, 1:KDP + 1]

    sqx = jnp.sum(x * x, axis=-1)
    dx2 = (sqx[:, :, None] + sqx[:, None, :]
           - 2.0 * jnp.einsum('bnd,bmd->bnm', x, x))
    dx = jnp.sqrt(jnp.maximum(dx2, 1e-12))
    idx_feat = jax.lax.top_k(-dx, KD + 1)[1][:, :, 1:]
    idx_out = jnp.concatenate([idx_feat, idx_pos], axis=2)

    R = B * N
    idx_l_flat = idx_l.reshape(R, ALLK)
    pos_flat = pos.reshape(R, 3)
    neigh = pos_flat[idx_l_flat]
    cents = _fps(neigh[:, :, 0], neigh[:, :, 1], neigh[:, :, 2],
                 KFPS, 2048)
    idx_fps = jnp.take_along_axis(idx_l_flat, cents, axis=1)

    C2 = 2 * C
    x2d = x.reshape(R, C)
    A = Wd[:, :C]
    Bm = Wd[:, C:]
    wcat1 = jnp.concatenate([A.T, (Bm - A).T], axis=1)
    pq = _matmul(x2d, wcat1)
    px, qx = pq[:, :C2], pq[:, C2:]

    flat14 = (idx_out + jnp.arange(B)[:, None, None] * N).reshape(-1)
    g14 = px[flat14].reshape(R, 14 * C2)
    ymax, stat1 = _edge1(g14, qx, 14, C2, 512)

    gb1 = jnp.stack([g1, b1], axis=0)
    A1 = W1[:, :C2]
    B1 = W1[:, C2:]
    wcat2 = jnp.concatenate([A1.T, (B1 - A1).T], axis=1)
    pq2 = _fin1_mm(ymax, stat1, gb1, wcat2, float(R * 14))
    p2, q2 = pq2[:, :128], pq2[:, 128:]

    flat32 = (idx_fps.reshape(B, N, KFPS)
              + jnp.arange(B)[:, None, None] * N).reshape(-1)
    gp = p2[flat32].reshape(R, KFPS * 128)
    c1, s2stat1 = _s2a(gp, q2, KFPS, 128, 256)

    gb2 = jnp.stack([g2, b2], axis=0)
    rr, s2stat2 = _s2b(c1, s2stat1, gb2, W2.T, KFPS, 128,
                       float(R * KFPS), 256)

    gb3 = jnp.stack([g3, b3], axis=0)
    out = _fin2(rr, s2stat2, gb3, float(R * KFPS)).reshape(B, N, 128)
    return out, idx_out


# speedup vs baseline: 1.2901x; 1.0027x over previous
import functools

import jax
import jax.numpy as jnp
from jax.experimental import pallas as pl
from jax.experimental.pallas import tpu as pltpu

_BN_EPS = 1e-5
_NEG = 0.2


def _lrelu(v):
    return jnp.where(v >= 0, v, _NEG * v)


def _mm_kernel(x_ref, w_ref, o_ref):
    o_ref[...] = jnp.dot(x_ref[...], w_ref[...],
                         preferred_element_type=jnp.float32)


def _matmul(x, w):
    m, k = x.shape
    k2, n = w.shape
    return pl.pallas_call(
        _mm_kernel,
        out_shape=jax.ShapeDtypeStruct((m, n), jnp.float32),
    )(x, w)


def _edge1_kernel(g_ref, q_ref, ymax_ref, stat_ref, *, kk, cc):
    r = pl.program_id(0)
    q = q_ref[...]
    acc_s = jnp.zeros((1, cc), jnp.float32)
    acc_q = jnp.zeros((1, cc), jnp.float32)
    m = jnp.full(q.shape, -jnp.inf, jnp.float32)
    for k in range(kk):
        y = g_ref[:, k * cc:(k + 1) * cc] + q
        acc_s = acc_s + jnp.sum(y, axis=0, keepdims=True)
        acc_q = acc_q + jnp.sum(y * y, axis=0, keepdims=True)
        m = jnp.maximum(m, y)
    ymax_ref[...] = m

    @pl.when(r == 0)
    def _init():
        stat_ref[...] = jnp.zeros_like(stat_ref)

    stat_ref[0:1, :] = stat_ref[0:1, :] + acc_s
    stat_ref[1:2, :] = stat_ref[1:2, :] + acc_q


def _edge1(g, q, kk, cc, rows_tile):
    rows = g.shape[0]
    grid = rows // rows_tile
    return pl.pallas_call(
        functools.partial(_edge1_kernel, kk=kk, cc=cc),
        grid=(grid,),
        in_specs=[
            pl.BlockSpec((rows_tile, kk * cc), lambda i: (i, 0)),
            pl.BlockSpec((rows_tile, cc), lambda i: (i, 0)),
        ],
        out_specs=[
            pl.BlockSpec((rows_tile, cc), lambda i: (i, 0)),
            pl.BlockSpec((8, cc), lambda i: (0, 0)),
        ],
        out_shape=[
            jax.ShapeDtypeStruct((rows, cc), jnp.float32),
            jax.ShapeDtypeStruct((8, cc), jnp.float32),
        ],
    )(g, q)


def _fin1_mm_kernel(ymax_ref, stat_ref, gb_ref, w_ref, o_ref, *, cnt):
    mean = stat_ref[0:1, :] / cnt
    var = stat_ref[1:2, :] / cnt - mean * mean
    scale = gb_ref[0:1, :] * jax.lax.rsqrt(var + _BN_EPS)
    shift = gb_ref[1:2, :] - mean * scale
    o4 = _lrelu(ymax_ref[...] * scale + shift)
    o_ref[...] = jnp.dot(o4, w_ref[...], preferred_element_type=jnp.float32)


def _fin1_mm(ymax, stat, gb, w, cnt):
    rows, cc = ymax.shape
    n = w.shape[1]
    return pl.pallas_call(
        functools.partial(_fin1_mm_kernel, cnt=cnt),
        out_shape=jax.ShapeDtypeStruct((rows, n), jnp.float32),
    )(ymax, stat, gb, w)


def _s2a_kernel(gp_ref, q_ref, stat_ref, *, kk, cc):
    r = pl.program_id(0)
    q = q_ref[...]
    acc_s = jnp.zeros((1, cc), jnp.float32)
    acc_q = jnp.zeros((1, cc), jnp.float32)
    for k in range(kk):
        c = gp_ref[:, k * cc:(k + 1) * cc] + q
        acc_s = acc_s + jnp.sum(c, axis=0, keepdims=True)
        acc_q = acc_q + jnp.sum(c * c, axis=0, keepdims=True)

    @pl.when(r == 0)
    def _init():
        stat_ref[...] = jnp.zeros_like(stat_ref)

    stat_ref[0:1, :] = stat_ref[0:1, :] + acc_s
    stat_ref[1:2, :] = stat_ref[1:2, :] + acc_q


def _s2a(gp, q, kk, cc, rows_tile):
    rows = gp.shape[0]
    grid = rows // rows_tile
    return pl.pallas_call(
        functools.partial(_s2a_kernel, kk=kk, cc=cc),
        grid=(grid,),
        in_specs=[
            pl.BlockSpec((rows_tile, kk * cc), lambda i: (i, 0)),
            pl.BlockSpec((rows_tile, cc), lambda i: (i, 0)),
        ],
        out_specs=pl.BlockSpec((8, cc), lambda i: (0, 0)),
        out_shape=jax.ShapeDtypeStruct((8, cc), jnp.float32),
    )(gp, q)


def _s2b_kernel(gp_ref, q_ref, stat1_ref, gb2_ref, w2_ref, r_ref, stat2_ref,
                *, kk, cc, cnt1):
    r = pl.program_id(0)
    mean = stat1_ref[0:1, :] / cnt1
    var = stat1_ref[1:2, :] / cnt1 - mean * mean
    scale = gb2_ref[0:1, :] * jax.lax.rsqrt(var + _BN_EPS)
    shift = gb2_ref[1:2, :] - mean * scale
    w2 = w2_ref[...]
    q = q_ref[...]
    acc_s = jnp.zeros((1, cc), jnp.float32)
    acc_q = jnp.zeros((1, cc), jnp.float32)
    m = jnp.full((gp_ref.shape[0], cc), -jnp.inf, jnp.float32)
    for k in range(kk):
        h = _lrelu((gp_ref[:, k * cc:(k + 1) * cc] + q) * scale + shift)
        c2 = jnp.dot(h, w2, preferred_element_type=jnp.float32)
        acc_s = acc_s + jnp.sum(c2, axis=0, keepdims=True)
        acc_q = acc_q + jnp.sum(c2 * c2, axis=0, keepdims=True)
        m = jnp.maximum(m, c2)
    r_ref[...] = m

    @pl.when(r == 0)
    def _init():
        stat2_ref[...] = jnp.zeros_like(stat2_ref)

    stat2_ref[0:1, :] = stat2_ref[0:1, :] + acc_s
    stat2_ref[1:2, :] = stat2_ref[1:2, :] + acc_q


def _s2b(gp, q, stat1, gb2, w2t, kk, cc, cnt1, rows_tile):
    rows = gp.shape[0]
    grid = rows // rows_tile
    return pl.pallas_call(
        functools.partial(_s2b_kernel, kk=kk, cc=cc, cnt1=cnt1),
        grid=(grid,),
        in_specs=[
            pl.BlockSpec((rows_tile, kk * cc), lambda i: (i, 0)),
            pl.BlockSpec((rows_tile, cc), lambda i: (i, 0)),
            pl.BlockSpec((8, cc), lambda i: (0, 0)),
            pl.BlockSpec((2, cc), lambda i: (0, 0)),
            pl.BlockSpec((cc, cc), lambda i: (0, 0)),
        ],
        out_specs=[
            pl.BlockSpec((rows_tile, cc), lambda i: (i, 0)),
            pl.BlockSpec((8, cc), lambda i: (0, 0)),
        ],
        out_shape=[
            jax.ShapeDtypeStruct((rows, cc), jnp.float32),
            jax.ShapeDtypeStruct((8, cc), jnp.float32),
        ],
    )(gp, q, stat1, gb2, w2t)


def _fin2_kernel(r_ref, stat_ref, gb_ref, o_ref, *, cnt):
    mean = stat_ref[0:1, :] / cnt
    var = stat_ref[1:2, :] / cnt - mean * mean
    scale = gb_ref[0:1, :] * jax.lax.rsqrt(var + _BN_EPS)
    shift = gb_ref[1:2, :] - mean * scale
    o_ref[...] = _lrelu(r_ref[...] * scale + shift)


def _fin2(rr, stat, gb, cnt):
    return pl.pallas_call(
        functools.partial(_fin2_kernel, cnt=cnt),
        out_shape=jax.ShapeDtypeStruct(rr.shape, jnp.float32),
    )(rr, stat, gb)


def _fps_kernel(nx_ref, ny_ref, nz_ref, cents_ref, *, npoint, allk):
    nx = nx_ref[...]
    ny = ny_ref[...]
    nz = nz_ref[...]
    rows = nx.shape[0]
    lane = jax.lax.broadcasted_iota(jnp.int32, (rows, allk), 1).astype(jnp.float32)
    dist = jnp.full((rows, allk), 1e10, jnp.float32)
    sel = jnp.zeros((rows, 1), jnp.float32)
    for t in range(npoint):
        cents_ref[:, t:t + 1] = sel.astype(jnp.int32)
        oh = (lane == sel).astype(jnp.float32)
        cx = jnp.sum(nx * oh, axis=1, keepdims=True)
        cy = jnp.sum(ny * oh, axis=1, keepdims=True)
        cz = jnp.sum(nz * oh, axis=1, keepdims=True)
        dx = nx - cx
        dy = ny - cy
        dz = nz - cz
        d = dx * dx + dy * dy + dz * dz
        dist = jnp.minimum(dist, d)
        mx = jnp.max(dist, axis=1, keepdims=True)
        sel = jnp.min(jnp.where(dist == mx, lane, float(allk)),
                      axis=1, keepdims=True)


def _fps(nx, ny, nz, npoint, rows_tile):
    rows, allk = nx.shape
    grid = rows // rows_tile
    spec = pl.BlockSpec((rows_tile, allk), lambda i: (i, 0))
    return pl.pallas_call(
        functools.partial(_fps_kernel, npoint=npoint, allk=allk),
        grid=(grid,),
        in_specs=[spec, spec, spec],
        out_specs=pl.BlockSpec((rows_tile, npoint), lambda i: (i, 0)),
        out_shape=jax.ShapeDtypeStruct((rows, npoint), jnp.int32),
    )(nx, ny, nz)


def kernel(x, pos, Wd, g1, b1, W1, g2, b2, W2, g3, b3):
    B, N, C = x.shape
    ALLK, KD, KDP, KFPS = 128, 8, 6, 32

    sqp = jnp.sum(pos * pos, axis=-1)
    dp2 = (sqp[:, :, None] + sqp[:, None, :]
           - 2.0 * jnp.einsum('bnd,bmd->bnm', pos, pos))
    dp = jnp.sqrt(jnp.maximum(dp2, 1e-12))
    idx_l = jax.lax.top_k(-dp, ALLK)[1]
    idx_pos = idx_l[:, :, 1:KDP + 1]

    sqx = jnp.sum(x * x, axis=-1)
    dx2 = (sqx[:, :, None] + sqx[:, None, :]
           - 2.0 * jnp.einsum('bnd,bmd->bnm', x, x))
    dx = jnp.sqrt(jnp.maximum(dx2, 1e-12))
    idx_feat = jax.lax.top_k(-dx, KD + 1)[1][:, :, 1:]
    idx_out = jnp.concatenate([idx_feat, idx_pos], axis=2)

    R = B * N
    idx_l_flat = idx_l.reshape(R, ALLK)
    pos_flat = pos.reshape(R, 3)
    neigh = pos_flat[idx_l_flat]
    cents = _fps(neigh[:, :, 0], neigh[:, :, 1], neigh[:, :, 2],
                 KFPS, 2048)
    idx_fps = jnp.take_along_axis(idx_l_flat, cents, axis=1)

    C2 = 2 * C
    x2d = x.reshape(R, C)
    A = Wd[:, :C]
    Bm = Wd[:, C:]
    wcat1 = jnp.concatenate([A.T, (Bm - A).T], axis=1)
    pq = _matmul(x2d, wcat1)
    px, qx = pq[:, :C2], pq[:, C2:]

    flat14 = (idx_out + jnp.arange(B)[:, None, None] * N).reshape(-1)
    g14 = px[flat14].reshape(R, 14 * C2)
    ymax, stat1 = _edge1(g14, qx, 14, C2, 512)

    gb1 = jnp.stack([g1, b1], axis=0)
    A1 = W1[:, :C2]
    B1 = W1[:, C2:]
    wcat2 = jnp.concatenate([A1.T, (B1 - A1).T], axis=1)
    pq2 = _fin1_mm(ymax, stat1, gb1, wcat2, float(R * 14))
    p2, q2 = pq2[:, :128], pq2[:, 128:]

    flat32 = (idx_fps.reshape(B, N, KFPS)
              + jnp.arange(B)[:, None, None] * N).reshape(-1)
    gp = p2[flat32].reshape(R, KFPS * 128)
    s2stat1 = _s2a(gp, q2, KFPS, 128, 256)

    gb2 = jnp.stack([g2, b2], axis=0)
    rr, s2stat2 = _s2b(gp, q2, s2stat1, gb2, W2.T, KFPS, 128,
                       float(R * KFPS), 256)

    gb3 = jnp.stack([g3, b3], axis=0)
    out = _fin2(rr, s2stat2, gb3, float(R * KFPS)).reshape(B, N, 128)
    return out, idx_out


# speedup vs baseline: 1.6219x; 1.2572x over previous
import functools

import jax
import jax.numpy as jnp
from jax.experimental import pallas as pl
from jax.experimental.pallas import tpu as pltpu

_BN_EPS = 1e-5
_NEG = 0.2


def _lrelu(v):
    return jnp.where(v >= 0, v, _NEG * v)


def _mm_kernel(x_ref, w_ref, o_ref):
    o_ref[...] = jnp.dot(x_ref[...], w_ref[...],
                         preferred_element_type=jnp.float32)


def _matmul(x, w):
    m, k = x.shape
    k2, n = w.shape
    return pl.pallas_call(
        _mm_kernel,
        out_shape=jax.ShapeDtypeStruct((m, n), jnp.float32),
    )(x, w)


def _edge1_kernel(g_ref, q_ref, ymax_ref, stat_ref, *, kk, cc):
    r = pl.program_id(0)
    q = q_ref[...]
    acc_s = jnp.zeros((1, cc), jnp.float32)
    acc_q = jnp.zeros((1, cc), jnp.float32)
    m = jnp.full(q.shape, -jnp.inf, jnp.float32)
    for k in range(kk):
        y = g_ref[:, k * cc:(k + 1) * cc] + q
        acc_s = acc_s + jnp.sum(y, axis=0, keepdims=True)
        acc_q = acc_q + jnp.sum(y * y, axis=0, keepdims=True)
        m = jnp.maximum(m, y)
    ymax_ref[...] = m

    @pl.when(r == 0)
    def _init():
        stat_ref[...] = jnp.zeros_like(stat_ref)

    stat_ref[0:1, :] = stat_ref[0:1, :] + acc_s
    stat_ref[1:2, :] = stat_ref[1:2, :] + acc_q


def _edge1(g, q, kk, cc, rows_tile):
    rows = g.shape[0]
    grid = rows // rows_tile
    return pl.pallas_call(
        functools.partial(_edge1_kernel, kk=kk, cc=cc),
        grid=(grid,),
        in_specs=[
            pl.BlockSpec((rows_tile, kk * cc), lambda i: (i, 0)),
            pl.BlockSpec((rows_tile, cc), lambda i: (i, 0)),
        ],
        out_specs=[
            pl.BlockSpec((rows_tile, cc), lambda i: (i, 0)),
            pl.BlockSpec((8, cc), lambda i: (0, 0)),
        ],
        out_shape=[
            jax.ShapeDtypeStruct((rows, cc), jnp.float32),
            jax.ShapeDtypeStruct((8, cc), jnp.float32),
        ],
    )(g, q)


def _fin1_mm_kernel(ymax_ref, stat_ref, gb_ref, w_ref, o_ref, *, cnt):
    mean = stat_ref[0:1, :] / cnt
    var = stat_ref[1:2, :] / cnt - mean * mean
    scale = gb_ref[0:1, :] * jax.lax.rsqrt(var + _BN_EPS)
    shift = gb_ref[1:2, :] - mean * scale
    o4 = _lrelu(ymax_ref[...] * scale + shift)
    o_ref[...] = jnp.dot(o4, w_ref[...], preferred_element_type=jnp.float32)


def _fin1_mm(ymax, stat, gb, w, cnt):
    rows, cc = ymax.shape
    n = w.shape[1]
    return pl.pallas_call(
        functools.partial(_fin1_mm_kernel, cnt=cnt),
        out_shape=jax.ShapeDtypeStruct((rows, n), jnp.float32),
    )(ymax, stat, gb, w)


def _s2a_kernel(gp_ref, q_ref, stat_ref, *, kk, cc):
    r = pl.program_id(0)
    q = q_ref[...]
    acc_s = jnp.zeros((1, cc), jnp.float32)
    acc_q = jnp.zeros((1, cc), jnp.float32)
    for k in range(kk):
        c = gp_ref[:, k * cc:(k + 1) * cc] + q
        acc_s = acc_s + jnp.sum(c, axis=0, keepdims=True)
        acc_q = acc_q + jnp.sum(c * c, axis=0, keepdims=True)

    @pl.when(r == 0)
    def _init():
        stat_ref[...] = jnp.zeros_like(stat_ref)

    stat_ref[0:1, :] = stat_ref[0:1, :] + acc_s
    stat_ref[1:2, :] = stat_ref[1:2, :] + acc_q


def _s2a(gp, q, kk, cc, rows_tile):
    rows = gp.shape[0]
    grid = rows // rows_tile
    return pl.pallas_call(
        functools.partial(_s2a_kernel, kk=kk, cc=cc),
        grid=(grid,),
        in_specs=[
            pl.BlockSpec((rows_tile, kk * cc), lambda i: (i, 0)),
            pl.BlockSpec((rows_tile, cc), lambda i: (i, 0)),
        ],
        out_specs=pl.BlockSpec((8, cc), lambda i: (0, 0)),
        out_shape=jax.ShapeDtypeStruct((8, cc), jnp.float32),
    )(gp, q)


def _s2b_kernel(gp_ref, q_ref, stat1_ref, gb2_ref, w2_ref, r_ref, stat2_ref,
                *, kk, cc, cnt1):
    r = pl.program_id(0)
    mean = stat1_ref[0:1, :] / cnt1
    var = stat1_ref[1:2, :] / cnt1 - mean * mean
    scale = gb2_ref[0:1, :] * jax.lax.rsqrt(var + _BN_EPS)
    shift = gb2_ref[1:2, :] - mean * scale
    w2 = w2_ref[...]
    q = q_ref[...]
    acc_s = jnp.zeros((1, cc), jnp.float32)
    acc_q = jnp.zeros((1, cc), jnp.float32)
    m = jnp.full((gp_ref.shape[0], cc), -jnp.inf, jnp.float32)
    for k in range(kk):
        h = _lrelu((gp_ref[:, k * cc:(k + 1) * cc] + q) * scale + shift)
        c2 = jnp.dot(h, w2, preferred_element_type=jnp.float32)
        acc_s = acc_s + jnp.sum(c2, axis=0, keepdims=True)
        acc_q = acc_q + jnp.sum(c2 * c2, axis=0, keepdims=True)
        m = jnp.maximum(m, c2)
    r_ref[...] = m

    @pl.when(r == 0)
    def _init():
        stat2_ref[...] = jnp.zeros_like(stat2_ref)

    stat2_ref[0:1, :] = stat2_ref[0:1, :] + acc_s
    stat2_ref[1:2, :] = stat2_ref[1:2, :] + acc_q


def _s2b(gp, q, stat1, gb2, w2t, kk, cc, cnt1, rows_tile):
    rows = gp.shape[0]
    grid = rows // rows_tile
    return pl.pallas_call(
        functools.partial(_s2b_kernel, kk=kk, cc=cc, cnt1=cnt1),
        grid=(grid,),
        in_specs=[
            pl.BlockSpec((rows_tile, kk * cc), lambda i: (i, 0)),
            pl.BlockSpec((rows_tile, cc), lambda i: (i, 0)),
            pl.BlockSpec((8, cc), lambda i: (0, 0)),
            pl.BlockSpec((2, cc), lambda i: (0, 0)),
            pl.BlockSpec((cc, cc), lambda i: (0, 0)),
        ],
        out_specs=[
            pl.BlockSpec((rows_tile, cc), lambda i: (i, 0)),
            pl.BlockSpec((8, cc), lambda i: (0, 0)),
        ],
        out_shape=[
            jax.ShapeDtypeStruct((rows, cc), jnp.float32),
            jax.ShapeDtypeStruct((8, cc), jnp.float32),
        ],
    )(gp, q, stat1, gb2, w2t)


def _fin2_kernel(r_ref, stat_ref, gb_ref, o_ref, *, cnt):
    mean = stat_ref[0:1, :] / cnt
    var = stat_ref[1:2, :] / cnt - mean * mean
    scale = gb_ref[0:1, :] * jax.lax.rsqrt(var + _BN_EPS)
    shift = gb_ref[1:2, :] - mean * scale
    o_ref[...] = _lrelu(r_ref[...] * scale + shift)


def _fin2(rr, stat, gb, cnt):
    return pl.pallas_call(
        functools.partial(_fin2_kernel, cnt=cnt),
        out_shape=jax.ShapeDtypeStruct(rr.shape, jnp.float32),
    )(rr, stat, gb)


def _knn_small_kernel(xr_ref, xt_ref, o_ref, *, nsel, n):
    xr = xr_ref[...]
    xt = xt_ref[0]
    rows = xr.shape[0]
    sqr = jnp.sum(xr * xr, axis=1, keepdims=True)
    sqc = jnp.sum(xt * xt, axis=0, keepdims=True)
    d2 = sqr + sqc - 2.0 * jnp.dot(xr, xt, preferred_element_type=jnp.float32)
    key = jnp.maximum(d2, 1e-12)
    lane = jax.lax.broadcasted_iota(jnp.int32, (rows, n), 1).astype(jnp.float32)
    for t in range(nsel):
        mn = jnp.min(key, axis=1, keepdims=True)
        sel = jnp.min(jnp.where(key == mn, lane, float(n)),
                      axis=1, keepdims=True)
        o_ref[:, t:t + 1] = sel.astype(jnp.int32)
        key = jnp.where(lane == sel, 1e30, key)


def _knn_small(x2d, xtb, nsel, rows_tile):
    rows, c = x2d.shape
    bb, _, n = xtb.shape
    grid = rows // rows_tile
    per_b = (rows // bb) // rows_tile
    return pl.pallas_call(
        functools.partial(_knn_small_kernel, nsel=nsel, n=n),
        grid=(grid,),
        in_specs=[
            pl.BlockSpec((rows_tile, c), lambda i: (i, 0)),
            pl.BlockSpec((1, c, n), lambda i: (i // per_b, 0, 0)),
        ],
        out_specs=pl.BlockSpec((rows_tile, nsel), lambda i: (i, 0)),
        out_shape=jax.ShapeDtypeStruct((rows, nsel), jnp.int32),
    )(x2d, xtb)


def _fps_kernel(nx_ref, ny_ref, nz_ref, cents_ref, *, npoint, allk):
    nx = nx_ref[...]
    ny = ny_ref[...]
    nz = nz_ref[...]
    rows = nx.shape[0]
    lane = jax.lax.broadcasted_iota(jnp.int32, (rows, allk), 1).astype(jnp.float32)
    dist = jnp.full((rows, allk), 1e10, jnp.float32)
    sel = jnp.zeros((rows, 1), jnp.float32)
    for t in range(npoint):
        cents_ref[:, t:t + 1] = sel.astype(jnp.int32)
        oh = (lane == sel).astype(jnp.float32)
        cx = jnp.sum(nx * oh, axis=1, keepdims=True)
        cy = jnp.sum(ny * oh, axis=1, keepdims=True)
        cz = jnp.sum(nz * oh, axis=1, keepdims=True)
        dx = nx - cx
        dy = ny - cy
        dz = nz - cz
        d = dx * dx + dy * dy + dz * dz
        dist = jnp.minimum(dist, d)
        mx = jnp.max(dist, axis=1, keepdims=True)
        sel = jnp.min(jnp.where(dist == mx, lane, float(allk)),
                      axis=1, keepdims=True)


def _fps(nx, ny, nz, npoint, rows_tile):
    rows, allk = nx.shape
    grid = rows // rows_tile
    spec = pl.BlockSpec((rows_tile, allk), lambda i: (i, 0))
    return pl.pallas_call(
        functools.partial(_fps_kernel, npoint=npoint, allk=allk),
        grid=(grid,),
        in_specs=[spec, spec, spec],
        out_specs=pl.BlockSpec((rows_tile, npoint), lambda i: (i, 0)),
        out_shape=jax.ShapeDtypeStruct((rows, npoint), jnp.int32),
    )(nx, ny, nz)


def kernel(x, pos, Wd, g1, b1, W1, g2, b2, W2, g3, b3):
    B, N, C = x.shape
    ALLK, KD, KDP, KFPS = 128, 8, 6, 32

    sqp = jnp.sum(pos * pos, axis=-1)
    dp2 = (sqp[:, :, None] + sqp[:, None, :]
           - 2.0 * jnp.einsum('bnd,bmd->bnm', pos, pos))
    dp = jnp.sqrt(jnp.maximum(dp2, 1e-12))
    idx_l = jax.lax.top_k(-dp, ALLK)[1]
    idx_pos = idx_l[:, :, 1:KDP + 1]

    xtb = jnp.transpose(x, (0, 2, 1))
    idx_feat = _knn_small(x.reshape(B * N, C), xtb, KD + 1,
                          256)[:, 1:].reshape(B, N, KD)
    idx_out = jnp.concatenate([idx_feat, idx_pos], axis=2)

    R = B * N
    idx_l_flat = idx_l.reshape(R, ALLK)
    pos_flat = pos.reshape(R, 3)
    neigh = pos_flat[idx_l_flat]
    cents = _fps(neigh[:, :, 0], neigh[:, :, 1], neigh[:, :, 2],
                 KFPS, 2048)
    idx_fps = jnp.take_along_axis(idx_l_flat, cents, axis=1)

    C2 = 2 * C
    x2d = x.reshape(R, C)
    A = Wd[:, :C]
    Bm = Wd[:, C:]
    wcat1 = jnp.concatenate([A.T, (Bm - A).T], axis=1)
    pq = _matmul(x2d, wcat1)
    px, qx = pq[:, :C2], pq[:, C2:]

    flat14 = (idx_out + jnp.arange(B)[:, None, None] * N).reshape(-1)
    g14 = px[flat14].reshape(R, 14 * C2)
    ymax, stat1 = _edge1(g14, qx, 14, C2, 512)

    gb1 = jnp.stack([g1, b1], axis=0)
    A1 = W1[:, :C2]
    B1 = W1[:, C2:]
    wcat2 = jnp.concatenate([A1.T, (B1 - A1).T], axis=1)
    pq2 = _fin1_mm(ymax, stat1, gb1, wcat2, float(R * 14))
    p2, q2 = pq2[:, :128], pq2[:, 128:]

    flat32 = (idx_fps.reshape(B, N, KFPS)
              + jnp.arange(B)[:, None, None] * N).reshape(-1)
    gp = p2[flat32].reshape(R, KFPS * 128)
    s2stat1 = _s2a(gp, q2, KFPS, 128, 256)

    gb2 = jnp.stack([g2, b2], axis=0)
    rr, s2stat2 = _s2b(gp, q2, s2stat1, gb2, W2.T, KFPS, 128,
                       float(R * KFPS), 256)

    gb3 = jnp.stack([g3, b3], axis=0)
    out = _fin2(rr, s2stat2, gb3, float(R * KFPS)).reshape(B, N, 128)
    return out, idx_out
